# Initial kernel scaffold; baseline (speedup 1.0000x reference)
import jax
import jax.numpy as jnp
from jax.experimental import pallas as pl


def kernel(rel_logits, obj_logits, rel_pair_idx, box, img_size):
    raise NotImplementedError("write your pallas kernel here")


# speedup vs baseline: 1.4372x; 1.4372x over previous
import jax
import jax.numpy as jnp
from jax import lax
from jax.experimental import pallas as pl
from jax.experimental.pallas import tpu as pltpu
from jax.experimental.pallas import tpu_sc as plsc

N_REL = 200000
N_OBJ = 20000
R_CLS = 51
O_CLS = 151
PACK_W = 128

SC_NC = 2
SC_NS = 16
SC_NW = SC_NC * SC_NS

REL_PAD = 200704
PER_W = REL_PAD // SC_NW
CH = 128
NCH = PER_W // CH
NCH_P = 56
G_CH = 64
G_NCH = PER_W // G_CH
G_NCH_P = 104

SORT_N = 262144
SORT_R = 2048
SORT_C = 128


def _xla_row_sum(e, P):
    n = e.shape[1]
    E = jnp.concatenate(
        [e, jnp.zeros((e.shape[0], P - n), jnp.float32)], axis=1
    )
    A = E
    for v in range(1, P // 8):
        A = A + pltpu.roll(E, P - 8 * v, axis=1)
    B = A + pltpu.roll(A, P - 4, axis=1)
    C = B + pltpu.roll(B, P - 2, axis=1)
    D = C + pltpu.roll(C, P - 1, axis=1)
    return D[:, 0:1]


def _obj_body(logits_ref, prob_ref, scores_ref, pred_ref):
    x = logits_ref[...]
    m = jnp.max(x, axis=1, keepdims=True)
    e = jnp.exp(x - m)
    z = _xla_row_sum(e, 256)
    p = e / z
    pfg = p[:, : O_CLS - 1]
    s = jnp.max(pfg, axis=1)
    col = lax.broadcasted_iota(jnp.int32, pfg.shape, 1)
    pred = jnp.min(jnp.where(pfg >= s[:, None], col, jnp.int32(2**30)), axis=1)
    prob_ref[:, : O_CLS - 1] = pfg
    prob_ref[:, O_CLS - 1 :] = jnp.zeros((x.shape[0], 1), jnp.float32)
    scores_ref[...] = s
    pred_ref[...] = pred


def _obj_call(obj_logits):
    blk = 1024
    grid = (N_OBJ + blk - 1) // blk
    return pl.pallas_call(
        _obj_body,
        grid=(grid,),
        in_specs=[pl.BlockSpec((blk, O_CLS), lambda i: (i, 0))],
        out_specs=[
            pl.BlockSpec((blk, O_CLS), lambda i: (i, 0)),
            pl.BlockSpec((blk,), lambda i: (i,)),
            pl.BlockSpec((blk,), lambda i: (i,)),
        ],
        out_shape=[
            jax.ShapeDtypeStruct((N_OBJ, O_CLS), jnp.float32),
            jax.ShapeDtypeStruct((N_OBJ,), jnp.float32),
            jax.ShapeDtypeStruct((N_OBJ,), jnp.int32),
        ],
    )(obj_logits)


def _rel_body(logits_ref, pairs_ref, packed_ref, scores_ref, cls_ref,
              i0_ref, i1_ref):
    x = logits_ref[...]
    rows = x.shape[0]
    m = jnp.max(x, axis=1, keepdims=True)
    e = jnp.exp(x - m)
    z = _xla_row_sum(e, 128)
    p = e / z
    i0 = pairs_ref[:, 0:1]
    i1 = pairs_ref[:, 1:2]
    packed_ref[:, :R_CLS] = p
    packed_ref[:, R_CLS : R_CLS + 1] = lax.bitcast_convert_type(i0, jnp.float32)
    packed_ref[:, R_CLS + 1 : R_CLS + 2] = lax.bitcast_convert_type(
        i1, jnp.float32
    )
    packed_ref[:, R_CLS + 2 :] = jnp.zeros(
        (rows, PACK_W - R_CLS - 2), jnp.float32
    )
    pfg = p[:, : R_CLS - 1]
    smax = jnp.max(pfg, axis=1, keepdims=True)
    scores_ref[...] = smax[:, 0]
    col = lax.broadcasted_iota(jnp.int32, pfg.shape, 1)
    cls_ref[...] = jnp.min(jnp.where(pfg >= smax, col, jnp.int32(2**30)), axis=1)
    i0_ref[...] = i0[:, 0]
    i1_ref[...] = i1[:, 0]


def _rel_call(rel_logits, rel_pair_idx):
    blk = 1024
    grid = (N_REL + blk - 1) // blk
    return pl.pallas_call(
        _rel_body,
        grid=(grid,),
        in_specs=[
            pl.BlockSpec((blk, R_CLS), lambda i: (i, 0)),
            pl.BlockSpec((blk, 2), lambda i: (i, 0)),
        ],
        out_specs=[
            pl.BlockSpec((blk, PACK_W), lambda i: (i, 0)),
            pl.BlockSpec((blk,), lambda i: (i,)),
            pl.BlockSpec((blk,), lambda i: (i,)),
            pl.BlockSpec((blk,), lambda i: (i,)),
            pl.BlockSpec((blk,), lambda i: (i,)),
        ],
        out_shape=[
            jax.ShapeDtypeStruct((N_REL, PACK_W), jnp.float32),
            jax.ShapeDtypeStruct((N_REL,), jnp.float32),
            jax.ShapeDtypeStruct((N_REL,), jnp.int32),
            jax.ShapeDtypeStruct((N_REL,), jnp.int32),
            jax.ShapeDtypeStruct((N_REL,), jnp.int32),
        ],
    )(rel_logits, rel_pair_idx)


def _triple_body(i0_hbm, i1_hbm, s_hbm, obj_hbm, out_hbm,
                 i0v, i1v, sbuf, s0b, s1b, tbuf, sem0, sem1):
    w = lax.axis_index("s") * SC_NC + lax.axis_index("c")
    base = w * PER_W
    pltpu.sync_copy(i0_hbm.at[pl.ds(w * NCH_P, NCH_P)], i0v)
    pltpu.sync_copy(i1_hbm.at[pl.ds(w * NCH_P, NCH_P)], i1v)
    pltpu.sync_copy(s_hbm.at[pl.ds(base, PER_W)], sbuf)

    def step(c, _):
        d0 = pltpu.async_copy(obj_hbm.at[i0v.at[c]], s0b, sem0)
        d1 = pltpu.async_copy(obj_hbm.at[i1v.at[c]], s1b, sem1)
        d0.wait()
        d1.wait()

        def vstep(v, _):
            o = c * CH + v * 16
            tbuf[pl.ds(o, 16)] = (
                sbuf[pl.ds(o, 16)]
                * s0b[pl.ds(v * 16, 16)]
                * s1b[pl.ds(v * 16, 16)]
            )
            return 0

        lax.fori_loop(0, CH // 16, vstep, 0)
        return 0

    lax.fori_loop(0, NCH, step, 0)
    pltpu.sync_copy(tbuf, out_hbm.at[pl.ds(base, PER_W)])


def _triple_call(obj_scores, rel_scores_p, i0r, i1r):
    mesh = plsc.VectorSubcoreMesh(
        core_axis_name="c", subcore_axis_name="s",
        num_cores=SC_NC, num_subcores=SC_NS,
    )
    fn = pl.kernel(
        _triple_body,
        out_type=jax.ShapeDtypeStruct((REL_PAD,), jnp.float32),
        mesh=mesh,
        scratch_types=[
            pltpu.VMEM((NCH_P, CH), jnp.int32),
            pltpu.VMEM((NCH_P, CH), jnp.int32),
            pltpu.VMEM((PER_W,), jnp.float32),
            pltpu.VMEM((CH,), jnp.float32),
            pltpu.VMEM((CH,), jnp.float32),
            pltpu.VMEM((PER_W,), jnp.float32),
            pltpu.SemaphoreType.DMA,
            pltpu.SemaphoreType.DMA,
        ],
    )
    return fn(i0r, i1r, rel_scores_p, obj_scores)


def _make_sort_tables():
    js, ks = [], []
    k = 2
    while k <= SORT_N:
        j = k // 2
        while j > 0:
            js.append(j)
            ks.append(k)
            j //= 2
        k *= 2
    return js, ks


def _sort_body(jt_ref, kt_ref, keys_any, out_any, ks, isc, ia, sem):
    p = pl.program_id(0)
    npass = pl.num_programs(0)

    @pl.when(p == 0)
    def _init():
        pltpu.make_async_copy(keys_any, ks, sem).start()
        ia[...] = (
            lax.broadcasted_iota(jnp.int32, (SORT_R, SORT_C), 0) * SORT_C
            + lax.broadcasted_iota(jnp.int32, (SORT_R, SORT_C), 1)
        )
        pltpu.make_async_copy(keys_any, ks, sem).wait()
        isc[...] = ia[...]

    j = jt_ref[p]
    k = kt_ref[p]
    kv = ks[...]
    iv = isc[...]
    iav = ia[...]
    low = (iav & j) == 0
    asc = (iav & k) == 0

    def lane_case(_):
        kp = jnp.where(
            low,
            pltpu.roll(kv, SORT_C - j, axis=1),
            pltpu.roll(kv, j, axis=1),
        )
        ip = jnp.where(
            low,
            pltpu.roll(iv, SORT_C - j, axis=1),
            pltpu.roll(iv, j, axis=1),
        )
        return kp, ip

    def row_case(_):
        jr = j // SORT_C
        kp = jnp.where(
            low,
            pltpu.roll(kv, SORT_R - jr, axis=0),
            pltpu.roll(kv, jr, axis=0),
        )
        ip = jnp.where(
            low,
            pltpu.roll(iv, SORT_R - jr, axis=0),
            pltpu.roll(iv, jr, axis=0),
        )
        return kp, ip

    kp, ip = lax.cond(j < SORT_C, lane_case, row_case, operand=None)

    prec = (kv > kp) | ((kv == kp) & (iv < ip))
    wantmin = low == asc
    take = prec == wantmin
    ks[...] = jnp.where(take, kv, kp)
    isc[...] = jnp.where(take, iv, ip)

    @pl.when(p == npass - 1)
    def _fin():
        pltpu.make_async_copy(isc, out_any, sem).start()
        pltpu.make_async_copy(isc, out_any, sem).wait()


def _sort_call(keys_i2d):
    js, ks = _make_sort_tables()
    npass = len(js)
    jt = jnp.asarray(js, dtype=jnp.int32)
    kt = jnp.asarray(ks, dtype=jnp.int32)
    return pl.pallas_call(
        _sort_body,
        grid=(npass,),
        in_specs=[
            pl.BlockSpec(memory_space=pltpu.SMEM),
            pl.BlockSpec(memory_space=pltpu.SMEM),
            pl.BlockSpec(memory_space=pl.ANY),
        ],
        out_specs=pl.BlockSpec(memory_space=pl.ANY),
        out_shape=jax.ShapeDtypeStruct((SORT_R, SORT_C), jnp.int32),
        scratch_shapes=[
            pltpu.VMEM((SORT_R, SORT_C), jnp.int32),
            pltpu.VMEM((SORT_R, SORT_C), jnp.int32),
            pltpu.VMEM((SORT_R, SORT_C), jnp.int32),
            pltpu.SemaphoreType.DMA,
        ],
        compiler_params=pltpu.CompilerParams(
            dimension_semantics=("arbitrary",),
        ),
    )(jt, kt, keys_i2d)


def _gather_body(sidx_hbm, packed_hbm, cls_hbm, packed_out, cls_out,
                 idxv, rowb, labb, sem0, sem1):
    w = lax.axis_index("s") * SC_NC + lax.axis_index("c")

    pltpu.sync_copy(sidx_hbm.at[pl.ds(w * G_NCH_P, G_NCH_P)], idxv)

    def step(c, _):
        ix = idxv.at[c]
        d0 = pltpu.async_copy(packed_hbm.at[ix], rowb, sem0)
        d1 = pltpu.async_copy(cls_hbm.at[ix], labb, sem1)
        d0.wait()
        d1.wait()
        o = w * PER_W + c * G_CH
        pltpu.sync_copy(rowb, packed_out.at[pl.ds(o, G_CH)])
        pltpu.sync_copy(labb, cls_out.at[pl.ds(o, G_CH)])
        return 0

    lax.fori_loop(0, G_NCH, step, 0)


def _gather_call(sidx2d, packed, rel_class):
    mesh = plsc.VectorSubcoreMesh(
        core_axis_name="c", subcore_axis_name="s",
        num_cores=SC_NC, num_subcores=SC_NS,
    )
    fn = pl.kernel(
        _gather_body,
        out_type=[
            jax.ShapeDtypeStruct((REL_PAD, PACK_W), jnp.float32),
            jax.ShapeDtypeStruct((REL_PAD,), jnp.int32),
        ],
        mesh=mesh,
        scratch_types=[
            pltpu.VMEM((G_NCH_P, G_CH), jnp.int32),
            pltpu.VMEM((G_CH, PACK_W), jnp.float32),
            pltpu.VMEM((G_CH,), jnp.int32),
            pltpu.SemaphoreType.DMA,
            pltpu.SemaphoreType.DMA,
        ],
    )
    return fn(sidx2d, packed, rel_class)


def _unpack_body(packed_ref, prob_ref, pairs_ref):
    blkrow = packed_ref[...]
    prob_ref[...] = blkrow[:, :R_CLS]
    pairs_ref[...] = lax.bitcast_convert_type(
        blkrow[:, R_CLS : R_CLS + 2], jnp.int32
    )


def _unpack_call(packed_s):
    blk = 4096
    grid = REL_PAD // blk
    return pl.pallas_call(
        _unpack_body,
        grid=(grid,),
        in_specs=[pl.BlockSpec((blk, PACK_W), lambda i: (i, 0))],
        out_specs=[
            pl.BlockSpec((blk, R_CLS), lambda i: (i, 0)),
            pl.BlockSpec((blk, 2), lambda i: (i, 0)),
        ],
        out_shape=[
            jax.ShapeDtypeStruct((N_REL, R_CLS), jnp.float32),
            jax.ShapeDtypeStruct((N_REL, 2), jnp.int32),
        ],
    )(packed_s)


def kernel(rel_logits, obj_logits, rel_pair_idx, box, img_size):
    obj_class_prob, obj_scores, obj_pred = _obj_call(obj_logits)
    packed, rel_scores, rel_class, i0, i1 = _rel_call(rel_logits, rel_pair_idx)

    pad = REL_PAD - N_REL
    rel_scores_p = jnp.pad(rel_scores, (0, pad))
    def _chunk_rows(x, nch, nch_p, ch):
        x = x.reshape(SC_NW, nch * ch)
        x = jnp.pad(x, ((0, 0), (0, (nch_p - nch) * ch)))
        return x.reshape(SC_NW * nch_p, ch)

    i0r = _chunk_rows(jnp.pad(i0, (0, pad)), NCH, NCH_P, CH)
    i1r = _chunk_rows(jnp.pad(i1, (0, pad)), NCH, NCH_P, CH)
    triple_p = _triple_call(obj_scores, rel_scores_p, i0r, i1r)

    keys = jnp.pad(triple_p, (0, SORT_N - REL_PAD))
    keys_i2d = lax.bitcast_convert_type(keys, jnp.int32).reshape(
        SORT_R, SORT_C
    )
    sidx = _sort_call(keys_i2d).reshape(-1)[:N_REL]
    sidx2d = _chunk_rows(jnp.pad(sidx, (0, pad)), G_NCH, G_NCH_P, G_CH)

    packed_s, labels_p = _gather_call(sidx2d, packed, rel_class)
    rel_prob_s, pairs_s = _unpack_call(packed_s)
    labels_s = labels_p[:N_REL]

    return (box, obj_pred, obj_scores, obj_class_prob,
            pairs_s, rel_prob_s, labels_s)


# speedup vs baseline: 1.6468x; 1.1459x over previous
import jax
import jax.numpy as jnp
from jax import lax
from jax.experimental import pallas as pl
from jax.experimental.pallas import tpu as pltpu
from jax.experimental.pallas import tpu_sc as plsc

N_REL = 200000
N_OBJ = 20000
R_CLS = 51
O_CLS = 151
PACK_W = 128

SC_NC = 2
SC_NS = 16
SC_NW = SC_NC * SC_NS

REL_PAD = 200704
PER_W = REL_PAD // SC_NW
CH = 128
NCH = PER_W // CH
NCH_P = 56
G_CH = 64
G_NCH = PER_W // G_CH
G_NCH_P = 104

SORT_N = 262144
SORT_R = 2048
SORT_C = 128


def _xla_row_sum(e, P):
    n = e.shape[1]
    E = jnp.concatenate(
        [e, jnp.zeros((e.shape[0], P - n), jnp.float32)], axis=1
    )
    A = E
    for v in range(1, (n + 7) // 8):
        A = A + pltpu.roll(E, P - 8 * v, axis=1)
    B = A + pltpu.roll(A, P - 4, axis=1)
    C = B + pltpu.roll(B, P - 2, axis=1)
    D = C + pltpu.roll(C, P - 1, axis=1)
    return D[:, 0:1]


def _obj_body(logits_ref, prob_ref, scores_ref, pred_ref):
    x = logits_ref[...]
    m = jnp.max(x, axis=1, keepdims=True)
    e = jnp.exp(x - m)
    z = _xla_row_sum(e, 256)
    p = e / z
    pfg = p[:, : O_CLS - 1]
    s = jnp.max(pfg, axis=1)
    col = lax.broadcasted_iota(jnp.int32, pfg.shape, 1)
    pred = jnp.min(jnp.where(pfg >= s[:, None], col, jnp.int32(2**30)), axis=1)
    prob_ref[:, : O_CLS - 1] = pfg
    prob_ref[:, O_CLS - 1 :] = jnp.zeros((x.shape[0], 1), jnp.float32)
    scores_ref[...] = s
    pred_ref[...] = pred


def _obj_call(obj_logits):
    blk = 2048
    grid = (N_OBJ + blk - 1) // blk
    return pl.pallas_call(
        _obj_body,
        grid=(grid,),
        in_specs=[pl.BlockSpec((blk, O_CLS), lambda i: (i, 0))],
        out_specs=[
            pl.BlockSpec((blk, O_CLS), lambda i: (i, 0)),
            pl.BlockSpec((blk,), lambda i: (i,)),
            pl.BlockSpec((blk,), lambda i: (i,)),
        ],
        out_shape=[
            jax.ShapeDtypeStruct((N_OBJ, O_CLS), jnp.float32),
            jax.ShapeDtypeStruct((N_OBJ,), jnp.float32),
            jax.ShapeDtypeStruct((N_OBJ,), jnp.int32),
        ],
    )(obj_logits)


def _rel_body(logits_ref, pairs_ref, packed_ref, scores_ref, cls_ref,
              i0_ref, i1_ref):
    x = logits_ref[...]
    rows = x.shape[0]
    m = jnp.max(x, axis=1, keepdims=True)
    e = jnp.exp(x - m)
    z = _xla_row_sum(e, 128)
    p = e / z
    i0 = pairs_ref[:, 0:1]
    i1 = pairs_ref[:, 1:2]
    packed_ref[:, :R_CLS] = p
    packed_ref[:, R_CLS : R_CLS + 1] = lax.bitcast_convert_type(i0, jnp.float32)
    packed_ref[:, R_CLS + 1 : R_CLS + 2] = lax.bitcast_convert_type(
        i1, jnp.float32
    )
    packed_ref[:, R_CLS + 2 :] = jnp.zeros(
        (rows, PACK_W - R_CLS - 2), jnp.float32
    )
    pfg = p[:, : R_CLS - 1]
    smax = jnp.max(pfg, axis=1, keepdims=True)
    scores_ref[...] = smax[:, 0]
    col = lax.broadcasted_iota(jnp.int32, pfg.shape, 1)
    cls_ref[...] = jnp.min(jnp.where(pfg >= smax, col, jnp.int32(2**30)), axis=1)
    i0_ref[...] = i0[:, 0]
    i1_ref[...] = i1[:, 0]


def _rel_call(rel_logits, rel_pair_idx):
    blk = 2048
    grid = (N_REL + blk - 1) // blk
    return pl.pallas_call(
        _rel_body,
        grid=(grid,),
        in_specs=[
            pl.BlockSpec((blk, R_CLS), lambda i: (i, 0)),
            pl.BlockSpec((blk, 2), lambda i: (i, 0)),
        ],
        out_specs=[
            pl.BlockSpec((blk, PACK_W), lambda i: (i, 0)),
            pl.BlockSpec((blk,), lambda i: (i,)),
            pl.BlockSpec((blk,), lambda i: (i,)),
            pl.BlockSpec((blk,), lambda i: (i,)),
            pl.BlockSpec((blk,), lambda i: (i,)),
        ],
        out_shape=[
            jax.ShapeDtypeStruct((N_REL, PACK_W), jnp.float32),
            jax.ShapeDtypeStruct((N_REL,), jnp.float32),
            jax.ShapeDtypeStruct((N_REL,), jnp.int32),
            jax.ShapeDtypeStruct((N_REL,), jnp.int32),
            jax.ShapeDtypeStruct((N_REL,), jnp.int32),
        ],
    )(rel_logits, rel_pair_idx)


def _triple_body(i0_hbm, i1_hbm, s_hbm, obj_hbm, out_hbm,
                 i0v, i1v, sbuf, s0b, s1b, tbuf, sem0, sem1):
    w = lax.axis_index("s") * SC_NC + lax.axis_index("c")
    base = w * PER_W
    pltpu.sync_copy(i0_hbm.at[pl.ds(w * NCH_P, NCH_P)], i0v)
    pltpu.sync_copy(i1_hbm.at[pl.ds(w * NCH_P, NCH_P)], i1v)
    pltpu.sync_copy(s_hbm.at[pl.ds(base, PER_W)], sbuf)

    def step(c, _):
        d0 = pltpu.async_copy(obj_hbm.at[i0v.at[c]], s0b, sem0)
        d1 = pltpu.async_copy(obj_hbm.at[i1v.at[c]], s1b, sem1)
        d0.wait()
        d1.wait()

        def vstep(v, _):
            o = c * CH + v * 16
            tbuf[pl.ds(o, 16)] = (
                sbuf[pl.ds(o, 16)]
                * s0b[pl.ds(v * 16, 16)]
                * s1b[pl.ds(v * 16, 16)]
            )
            return 0

        lax.fori_loop(0, CH // 16, vstep, 0)
        return 0

    lax.fori_loop(0, NCH, step, 0)
    pltpu.sync_copy(tbuf, out_hbm.at[pl.ds(base, PER_W)])


def _triple_call(obj_scores, rel_scores_p, i0r, i1r):
    mesh = plsc.VectorSubcoreMesh(
        core_axis_name="c", subcore_axis_name="s",
        num_cores=SC_NC, num_subcores=SC_NS,
    )
    fn = pl.kernel(
        _triple_body,
        out_type=jax.ShapeDtypeStruct((REL_PAD,), jnp.float32),
        mesh=mesh,
        scratch_types=[
            pltpu.VMEM((NCH_P, CH), jnp.int32),
            pltpu.VMEM((NCH_P, CH), jnp.int32),
            pltpu.VMEM((PER_W,), jnp.float32),
            pltpu.VMEM((CH,), jnp.float32),
            pltpu.VMEM((CH,), jnp.float32),
            pltpu.VMEM((PER_W,), jnp.float32),
            pltpu.SemaphoreType.DMA,
            pltpu.SemaphoreType.DMA,
        ],
    )
    return fn(i0r, i1r, rel_scores_p, obj_scores)


def _make_sort_tables():
    js, ks = [], []
    k = 2
    while k <= SORT_N:
        j = k // 2
        while j > 0:
            js.append(j)
            ks.append(k)
            j //= 2
        k *= 2
    return js, ks


def _sort_body(jt_ref, kt_ref, keys_any, out_any, ks, isc, ia, sem):
    p = pl.program_id(0)
    npass = pl.num_programs(0)

    @pl.when(p == 0)
    def _init():
        pltpu.make_async_copy(keys_any, ks, sem).start()
        ia[...] = (
            lax.broadcasted_iota(jnp.int32, (SORT_R, SORT_C), 0) * SORT_C
            + lax.broadcasted_iota(jnp.int32, (SORT_R, SORT_C), 1)
        )
        pltpu.make_async_copy(keys_any, ks, sem).wait()
        isc[...] = ia[...]

    j = jt_ref[p]
    k = kt_ref[p]
    kv = ks[...]
    iv = isc[...]
    iav = ia[...]
    low = (iav & j) == 0
    asc = (iav & k) == 0

    def lane_case(_):
        kp = jnp.where(
            low,
            pltpu.roll(kv, SORT_C - j, axis=1),
            pltpu.roll(kv, j, axis=1),
        )
        ip = jnp.where(
            low,
            pltpu.roll(iv, SORT_C - j, axis=1),
            pltpu.roll(iv, j, axis=1),
        )
        return kp, ip

    def row_case(_):
        jr = j // SORT_C
        kp = jnp.where(
            low,
            pltpu.roll(kv, SORT_R - jr, axis=0),
            pltpu.roll(kv, jr, axis=0),
        )
        ip = jnp.where(
            low,
            pltpu.roll(iv, SORT_R - jr, axis=0),
            pltpu.roll(iv, jr, axis=0),
        )
        return kp, ip

    kp, ip = lax.cond(j < SORT_C, lane_case, row_case, operand=None)

    prec = (kv > kp) | ((kv == kp) & (iv < ip))
    wantmin = low == asc
    take = prec == wantmin
    ks[...] = jnp.where(take, kv, kp)
    isc[...] = jnp.where(take, iv, ip)

    @pl.when(p == npass - 1)
    def _fin():
        pltpu.make_async_copy(isc, out_any, sem).start()
        pltpu.make_async_copy(isc, out_any, sem).wait()


def _sort_call(keys_i2d):
    js, ks = _make_sort_tables()
    npass = len(js)
    jt = jnp.asarray(js, dtype=jnp.int32)
    kt = jnp.asarray(ks, dtype=jnp.int32)
    return pl.pallas_call(
        _sort_body,
        grid=(npass,),
        in_specs=[
            pl.BlockSpec(memory_space=pltpu.SMEM),
            pl.BlockSpec(memory_space=pltpu.SMEM),
            pl.BlockSpec(memory_space=pl.ANY),
        ],
        out_specs=pl.BlockSpec(memory_space=pl.ANY),
        out_shape=jax.ShapeDtypeStruct((SORT_R, SORT_C), jnp.int32),
        scratch_shapes=[
            pltpu.VMEM((SORT_R, SORT_C), jnp.int32),
            pltpu.VMEM((SORT_R, SORT_C), jnp.int32),
            pltpu.VMEM((SORT_R, SORT_C), jnp.int32),
            pltpu.SemaphoreType.DMA,
        ],
        compiler_params=pltpu.CompilerParams(
            dimension_semantics=("arbitrary",),
        ),
    )(jt, kt, keys_i2d)


def _gather_body(sidx_hbm, packed_hbm, cls_hbm, packed_out, cls_out,
                 idxv, rowb, labb, sem0, sem1):
    w = lax.axis_index("s") * SC_NC + lax.axis_index("c")

    pltpu.sync_copy(sidx_hbm.at[pl.ds(w * G_NCH_P, G_NCH_P)], idxv)

    def step(c, _):
        ix = idxv.at[c]
        d0 = pltpu.async_copy(packed_hbm.at[ix], rowb, sem0)
        d1 = pltpu.async_copy(cls_hbm.at[ix], labb, sem1)
        d0.wait()
        d1.wait()
        o = w * PER_W + c * G_CH
        pltpu.sync_copy(rowb, packed_out.at[pl.ds(o, G_CH)])
        pltpu.sync_copy(labb, cls_out.at[pl.ds(o, G_CH)])
        return 0

    lax.fori_loop(0, G_NCH, step, 0)


def _gather_call(sidx2d, packed, rel_class):
    mesh = plsc.VectorSubcoreMesh(
        core_axis_name="c", subcore_axis_name="s",
        num_cores=SC_NC, num_subcores=SC_NS,
    )
    fn = pl.kernel(
        _gather_body,
        out_type=[
            jax.ShapeDtypeStruct((REL_PAD, PACK_W), jnp.float32),
            jax.ShapeDtypeStruct((REL_PAD,), jnp.int32),
        ],
        mesh=mesh,
        scratch_types=[
            pltpu.VMEM((G_NCH_P, G_CH), jnp.int32),
            pltpu.VMEM((G_CH, PACK_W), jnp.float32),
            pltpu.VMEM((G_CH,), jnp.int32),
            pltpu.SemaphoreType.DMA,
            pltpu.SemaphoreType.DMA,
        ],
    )
    return fn(sidx2d, packed, rel_class)


def _unpack_body(packed_ref, prob_ref, pairs_ref):
    blkrow = packed_ref[...]
    prob_ref[...] = blkrow[:, :R_CLS]
    pairs_ref[...] = lax.bitcast_convert_type(
        blkrow[:, R_CLS : R_CLS + 2], jnp.int32
    )


def _unpack_call(packed_s):
    blk = 4096
    grid = REL_PAD // blk
    return pl.pallas_call(
        _unpack_body,
        grid=(grid,),
        in_specs=[pl.BlockSpec((blk, PACK_W), lambda i: (i, 0))],
        out_specs=[
            pl.BlockSpec((blk, R_CLS), lambda i: (i, 0)),
            pl.BlockSpec((blk, 2), lambda i: (i, 0)),
        ],
        out_shape=[
            jax.ShapeDtypeStruct((N_REL, R_CLS), jnp.float32),
            jax.ShapeDtypeStruct((N_REL, 2), jnp.int32),
        ],
    )(packed_s)


def kernel(rel_logits, obj_logits, rel_pair_idx, box, img_size):
    obj_class_prob, obj_scores, obj_pred = _obj_call(obj_logits)
    packed, rel_scores, rel_class, i0, i1 = _rel_call(rel_logits, rel_pair_idx)

    pad = REL_PAD - N_REL
    rel_scores_p = jnp.pad(rel_scores, (0, pad))
    def _chunk_rows(x, nch, nch_p, ch):
        x = x.reshape(SC_NW, nch * ch)
        x = jnp.pad(x, ((0, 0), (0, (nch_p - nch) * ch)))
        return x.reshape(SC_NW * nch_p, ch)

    i0r = _chunk_rows(jnp.pad(i0, (0, pad)), NCH, NCH_P, CH)
    i1r = _chunk_rows(jnp.pad(i1, (0, pad)), NCH, NCH_P, CH)
    triple_p = _triple_call(obj_scores, rel_scores_p, i0r, i1r)

    keys = jnp.pad(triple_p, (0, SORT_N - REL_PAD))
    keys_i2d = lax.bitcast_convert_type(keys, jnp.int32).reshape(
        SORT_R, SORT_C
    )
    sidx = _sort_call(keys_i2d).reshape(-1)[:N_REL]
    sidx2d = _chunk_rows(jnp.pad(sidx, (0, pad)), G_NCH, G_NCH_P, G_CH)

    packed_s, labels_p = _gather_call(sidx2d, packed, rel_class)
    rel_prob_s, pairs_s = _unpack_call(packed_s)
    labels_s = labels_p[:N_REL]

    return (box, obj_pred, obj_scores, obj_class_prob,
            pairs_s, rel_prob_s, labels_s)


# speedup vs baseline: 1.8929x; 1.1494x over previous
import jax
import jax.numpy as jnp
from jax import lax
from jax.experimental import pallas as pl
from jax.experimental.pallas import tpu as pltpu
from jax.experimental.pallas import tpu_sc as plsc

N_REL = 200000
N_OBJ = 20000
R_CLS = 51
O_CLS = 151
PACK_W = 128

SC_NC = 2
SC_NS = 16
SC_NW = SC_NC * SC_NS

REL_PAD = 200704
PER_W = REL_PAD // SC_NW
CH = 128
NCH = PER_W // CH
NCH_P = 56
G_CH = 64
G_NCH = PER_W // G_CH
G_NCH_P = 104

SORT_N = 262144
SORT_R = 2048
SORT_C = 128


def _xla_row_sum(e, P):
    del P
    n = e.shape[1]
    nstrip = (n + 7) // 8
    E = jnp.concatenate(
        [e, jnp.zeros((e.shape[0], 8 * nstrip - n), jnp.float32)], axis=1
    )
    eT = E.T
    A = eT[0:8]
    for v in range(1, nstrip):
        A = A + eT[8 * v : 8 * v + 8]
    B = A[0:4] + A[4:8]
    C = B[0:2] + B[2:4]
    D = C[0:1] + C[1:2]
    return D.T


def _obj_body(logits_ref, prob_ref, scores_ref, pred_ref):
    x = logits_ref[...]
    m = jnp.max(x, axis=1, keepdims=True)
    e = jnp.exp(x - m)
    z = _xla_row_sum(e, 256)
    p = e / z
    pfg = p[:, : O_CLS - 1]
    s = jnp.max(pfg, axis=1)
    col = lax.broadcasted_iota(jnp.int32, pfg.shape, 1)
    pred = jnp.min(jnp.where(pfg >= s[:, None], col, jnp.int32(2**30)), axis=1)
    prob_ref[:, : O_CLS - 1] = pfg
    prob_ref[:, O_CLS - 1 :] = jnp.zeros((x.shape[0], 1), jnp.float32)
    scores_ref[...] = s
    pred_ref[...] = pred


def _obj_call(obj_logits):
    blk = 2048
    grid = (N_OBJ + blk - 1) // blk
    return pl.pallas_call(
        _obj_body,
        grid=(grid,),
        in_specs=[pl.BlockSpec((blk, O_CLS), lambda i: (i, 0))],
        out_specs=[
            pl.BlockSpec((blk, O_CLS), lambda i: (i, 0)),
            pl.BlockSpec((blk,), lambda i: (i,)),
            pl.BlockSpec((blk,), lambda i: (i,)),
        ],
        out_shape=[
            jax.ShapeDtypeStruct((N_OBJ, O_CLS), jnp.float32),
            jax.ShapeDtypeStruct((N_OBJ,), jnp.float32),
            jax.ShapeDtypeStruct((N_OBJ,), jnp.int32),
        ],
    )(obj_logits)


def _rel_body(logits_ref, pairs_ref, packed_ref, scores_ref, cls_ref,
              i0_ref, i1_ref):
    x = logits_ref[...]
    rows = x.shape[0]
    m = jnp.max(x, axis=1, keepdims=True)
    e = jnp.exp(x - m)
    z = _xla_row_sum(e, 128)
    p = e / z
    i0 = pairs_ref[:, 0:1]
    i1 = pairs_ref[:, 1:2]
    packed_ref[:, :R_CLS] = p
    packed_ref[:, R_CLS : R_CLS + 1] = lax.bitcast_convert_type(i0, jnp.float32)
    packed_ref[:, R_CLS + 1 : R_CLS + 2] = lax.bitcast_convert_type(
        i1, jnp.float32
    )
    packed_ref[:, R_CLS + 2 :] = jnp.zeros(
        (rows, PACK_W - R_CLS - 2), jnp.float32
    )
    pfg = p[:, : R_CLS - 1]
    smax = jnp.max(pfg, axis=1, keepdims=True)
    scores_ref[...] = smax[:, 0]
    col = lax.broadcasted_iota(jnp.int32, pfg.shape, 1)
    cls_ref[...] = jnp.min(jnp.where(pfg >= smax, col, jnp.int32(2**30)), axis=1)
    i0_ref[...] = i0[:, 0]
    i1_ref[...] = i1[:, 0]


def _rel_call(rel_logits, rel_pair_idx):
    blk = 2048
    grid = (N_REL + blk - 1) // blk
    return pl.pallas_call(
        _rel_body,
        grid=(grid,),
        in_specs=[
            pl.BlockSpec((blk, R_CLS), lambda i: (i, 0)),
            pl.BlockSpec((blk, 2), lambda i: (i, 0)),
        ],
        out_specs=[
            pl.BlockSpec((blk, PACK_W), lambda i: (i, 0)),
            pl.BlockSpec((blk,), lambda i: (i,)),
            pl.BlockSpec((blk,), lambda i: (i,)),
            pl.BlockSpec((blk,), lambda i: (i,)),
            pl.BlockSpec((blk,), lambda i: (i,)),
        ],
        out_shape=[
            jax.ShapeDtypeStruct((N_REL, PACK_W), jnp.float32),
            jax.ShapeDtypeStruct((N_REL,), jnp.float32),
            jax.ShapeDtypeStruct((N_REL,), jnp.int32),
            jax.ShapeDtypeStruct((N_REL,), jnp.int32),
            jax.ShapeDtypeStruct((N_REL,), jnp.int32),
        ],
    )(rel_logits, rel_pair_idx)


def _triple_body(i0_hbm, i1_hbm, s_hbm, obj_hbm, out_hbm,
                 i0v, i1v, sbuf, s0b, s1b, tbuf, sem0, sem1):
    w = lax.axis_index("s") * SC_NC + lax.axis_index("c")
    base = w * PER_W
    pltpu.sync_copy(i0_hbm.at[pl.ds(w * NCH_P, NCH_P)], i0v)
    pltpu.sync_copy(i1_hbm.at[pl.ds(w * NCH_P, NCH_P)], i1v)
    pltpu.sync_copy(s_hbm.at[pl.ds(base, PER_W)], sbuf)

    def step(c, _):
        d0 = pltpu.async_copy(obj_hbm.at[i0v.at[c]], s0b, sem0)
        d1 = pltpu.async_copy(obj_hbm.at[i1v.at[c]], s1b, sem1)
        d0.wait()
        d1.wait()

        def vstep(v, _):
            o = c * CH + v * 16
            tbuf[pl.ds(o, 16)] = (
                sbuf[pl.ds(o, 16)]
                * s0b[pl.ds(v * 16, 16)]
                * s1b[pl.ds(v * 16, 16)]
            )
            return 0

        lax.fori_loop(0, CH // 16, vstep, 0)
        return 0

    lax.fori_loop(0, NCH, step, 0)
    pltpu.sync_copy(tbuf, out_hbm.at[pl.ds(base, PER_W)])


def _triple_call(obj_scores, rel_scores_p, i0r, i1r):
    mesh = plsc.VectorSubcoreMesh(
        core_axis_name="c", subcore_axis_name="s",
        num_cores=SC_NC, num_subcores=SC_NS,
    )
    fn = pl.kernel(
        _triple_body,
        out_type=jax.ShapeDtypeStruct((REL_PAD,), jnp.float32),
        mesh=mesh,
        scratch_types=[
            pltpu.VMEM((NCH_P, CH), jnp.int32),
            pltpu.VMEM((NCH_P, CH), jnp.int32),
            pltpu.VMEM((PER_W,), jnp.float32),
            pltpu.VMEM((CH,), jnp.float32),
            pltpu.VMEM((CH,), jnp.float32),
            pltpu.VMEM((PER_W,), jnp.float32),
            pltpu.SemaphoreType.DMA,
            pltpu.SemaphoreType.DMA,
        ],
    )
    return fn(i0r, i1r, rel_scores_p, obj_scores)


def _make_sort_tables():
    js, ks = [], []
    k = 2
    while k <= SORT_N:
        j = k // 2
        while j > 0:
            js.append(j)
            ks.append(k)
            j //= 2
        k *= 2
    return js, ks


def _sort_body(jt_ref, kt_ref, keys_any, out_any, ks, isc, ia, sem):
    p = pl.program_id(0)
    npass = pl.num_programs(0)

    @pl.when(p == 0)
    def _init():
        pltpu.make_async_copy(keys_any, ks, sem).start()
        ia[...] = (
            lax.broadcasted_iota(jnp.int32, (SORT_R, SORT_C), 0) * SORT_C
            + lax.broadcasted_iota(jnp.int32, (SORT_R, SORT_C), 1)
        )
        pltpu.make_async_copy(keys_any, ks, sem).wait()
        isc[...] = ia[...]

    j = jt_ref[p]
    k = kt_ref[p]
    kv = ks[...]
    iv = isc[...]
    iav = ia[...]
    low = (iav & j) == 0
    asc = (iav & k) == 0

    def lane_case(_):
        kp = jnp.where(
            low,
            pltpu.roll(kv, SORT_C - j, axis=1),
            pltpu.roll(kv, j, axis=1),
        )
        ip = jnp.where(
            low,
            pltpu.roll(iv, SORT_C - j, axis=1),
            pltpu.roll(iv, j, axis=1),
        )
        return kp, ip

    def row_case(_):
        jr = j // SORT_C
        kp = jnp.where(
            low,
            pltpu.roll(kv, SORT_R - jr, axis=0),
            pltpu.roll(kv, jr, axis=0),
        )
        ip = jnp.where(
            low,
            pltpu.roll(iv, SORT_R - jr, axis=0),
            pltpu.roll(iv, jr, axis=0),
        )
        return kp, ip

    kp, ip = lax.cond(j < SORT_C, lane_case, row_case, operand=None)

    prec = (kv > kp) | ((kv == kp) & (iv < ip))
    wantmin = low == asc
    take = prec == wantmin
    ks[...] = jnp.where(take, kv, kp)
    isc[...] = jnp.where(take, iv, ip)

    @pl.when(p == npass - 1)
    def _fin():
        pltpu.make_async_copy(isc, out_any, sem).start()
        pltpu.make_async_copy(isc, out_any, sem).wait()


def _sort_call(keys_i2d):
    js, ks = _make_sort_tables()
    npass = len(js)
    jt = jnp.asarray(js, dtype=jnp.int32)
    kt = jnp.asarray(ks, dtype=jnp.int32)
    return pl.pallas_call(
        _sort_body,
        grid=(npass,),
        in_specs=[
            pl.BlockSpec(memory_space=pltpu.SMEM),
            pl.BlockSpec(memory_space=pltpu.SMEM),
            pl.BlockSpec(memory_space=pl.ANY),
        ],
        out_specs=pl.BlockSpec(memory_space=pl.ANY),
        out_shape=jax.ShapeDtypeStruct((SORT_R, SORT_C), jnp.int32),
        scratch_shapes=[
            pltpu.VMEM((SORT_R, SORT_C), jnp.int32),
            pltpu.VMEM((SORT_R, SORT_C), jnp.int32),
            pltpu.VMEM((SORT_R, SORT_C), jnp.int32),
            pltpu.SemaphoreType.DMA,
        ],
        compiler_params=pltpu.CompilerParams(
            dimension_semantics=("arbitrary",),
        ),
    )(jt, kt, keys_i2d)


def _gather_body(sidx_hbm, packed_hbm, cls_hbm, packed_out, cls_out,
                 idxv, rowb, labb, sem0, sem1):
    w = lax.axis_index("s") * SC_NC + lax.axis_index("c")

    pltpu.sync_copy(sidx_hbm.at[pl.ds(w * G_NCH_P, G_NCH_P)], idxv)

    def step(c, _):
        ix = idxv.at[c]
        d0 = pltpu.async_copy(packed_hbm.at[ix], rowb, sem0)
        d1 = pltpu.async_copy(cls_hbm.at[ix], labb, sem1)
        d0.wait()
        d1.wait()
        o = w * PER_W + c * G_CH
        pltpu.sync_copy(rowb, packed_out.at[pl.ds(o, G_CH)])
        pltpu.sync_copy(labb, cls_out.at[pl.ds(o, G_CH)])
        return 0

    lax.fori_loop(0, G_NCH, step, 0)


def _gather_call(sidx2d, packed, rel_class):
    mesh = plsc.VectorSubcoreMesh(
        core_axis_name="c", subcore_axis_name="s",
        num_cores=SC_NC, num_subcores=SC_NS,
    )
    fn = pl.kernel(
        _gather_body,
        out_type=[
            jax.ShapeDtypeStruct((REL_PAD, PACK_W), jnp.float32),
            jax.ShapeDtypeStruct((REL_PAD,), jnp.int32),
        ],
        mesh=mesh,
        scratch_types=[
            pltpu.VMEM((G_NCH_P, G_CH), jnp.int32),
            pltpu.VMEM((G_CH, PACK_W), jnp.float32),
            pltpu.VMEM((G_CH,), jnp.int32),
            pltpu.SemaphoreType.DMA,
            pltpu.SemaphoreType.DMA,
        ],
    )
    return fn(sidx2d, packed, rel_class)


def _unpack_body(packed_ref, prob_ref, pairs_ref):
    blkrow = packed_ref[...]
    prob_ref[...] = blkrow[:, :R_CLS]
    pairs_ref[...] = lax.bitcast_convert_type(
        blkrow[:, R_CLS : R_CLS + 2], jnp.int32
    )


def _unpack_call(packed_s):
    blk = 4096
    grid = REL_PAD // blk
    return pl.pallas_call(
        _unpack_body,
        grid=(grid,),
        in_specs=[pl.BlockSpec((blk, PACK_W), lambda i: (i, 0))],
        out_specs=[
            pl.BlockSpec((blk, R_CLS), lambda i: (i, 0)),
            pl.BlockSpec((blk, 2), lambda i: (i, 0)),
        ],
        out_shape=[
            jax.ShapeDtypeStruct((N_REL, R_CLS), jnp.float32),
            jax.ShapeDtypeStruct((N_REL, 2), jnp.int32),
        ],
    )(packed_s)


def kernel(rel_logits, obj_logits, rel_pair_idx, box, img_size):
    obj_class_prob, obj_scores, obj_pred = _obj_call(obj_logits)
    packed, rel_scores, rel_class, i0, i1 = _rel_call(rel_logits, rel_pair_idx)

    pad = REL_PAD - N_REL
    rel_scores_p = jnp.pad(rel_scores, (0, pad))
    def _chunk_rows(x, nch, nch_p, ch):
        x = x.reshape(SC_NW, nch * ch)
        x = jnp.pad(x, ((0, 0), (0, (nch_p - nch) * ch)))
        return x.reshape(SC_NW * nch_p, ch)

    i0r = _chunk_rows(jnp.pad(i0, (0, pad)), NCH, NCH_P, CH)
    i1r = _chunk_rows(jnp.pad(i1, (0, pad)), NCH, NCH_P, CH)
    triple_p = _triple_call(obj_scores, rel_scores_p, i0r, i1r)

    keys = jnp.pad(triple_p, (0, SORT_N - REL_PAD))
    keys_i2d = lax.bitcast_convert_type(keys, jnp.int32).reshape(
        SORT_R, SORT_C
    )
    sidx = _sort_call(keys_i2d).reshape(-1)[:N_REL]
    sidx2d = _chunk_rows(jnp.pad(sidx, (0, pad)), G_NCH, G_NCH_P, G_CH)

    packed_s, labels_p = _gather_call(sidx2d, packed, rel_class)
    rel_prob_s, pairs_s = _unpack_call(packed_s)
    labels_s = labels_p[:N_REL]

    return (box, obj_pred, obj_scores, obj_class_prob,
            pairs_s, rel_prob_s, labels_s)


# speedup vs baseline: 1.9347x; 1.0221x over previous
import jax
import jax.numpy as jnp
from jax import lax
from jax.experimental import pallas as pl
from jax.experimental.pallas import tpu as pltpu
from jax.experimental.pallas import tpu_sc as plsc

N_REL = 200000
N_OBJ = 20000
R_CLS = 51
O_CLS = 151
PACK_W = 128

SC_NC = 2
SC_NS = 16
SC_NW = SC_NC * SC_NS

REL_PAD = 200704
PER_W = REL_PAD // SC_NW
CH = 128
NCH = PER_W // CH
NCH_P = 56
G_CH = 128
G_NCH = PER_W // G_CH
G_NCH_P = 56

SORT_N = 262144
SORT_R = 2048
SORT_C = 128


def _xla_row_sum(e, P):
    del P
    n = e.shape[1]
    nstrip = (n + 7) // 8
    E = jnp.concatenate(
        [e, jnp.zeros((e.shape[0], 8 * nstrip - n), jnp.float32)], axis=1
    )
    eT = E.T
    A = eT[0:8]
    for v in range(1, nstrip):
        A = A + eT[8 * v : 8 * v + 8]
    B = A[0:4] + A[4:8]
    C = B[0:2] + B[2:4]
    D = C[0:1] + C[1:2]
    return D.T


def _obj_body(logits_ref, prob_ref, scores_ref, pred_ref):
    x = logits_ref[...]
    m = jnp.max(x, axis=1, keepdims=True)
    e = jnp.exp(x - m)
    z = _xla_row_sum(e, 256)
    p = e / z
    pfg = p[:, : O_CLS - 1]
    s = jnp.max(pfg, axis=1)
    col = lax.broadcasted_iota(jnp.int32, pfg.shape, 1)
    pred = jnp.min(jnp.where(pfg >= s[:, None], col, jnp.int32(2**30)), axis=1)
    prob_ref[:, : O_CLS - 1] = pfg
    prob_ref[:, O_CLS - 1 :] = jnp.zeros((x.shape[0], 1), jnp.float32)
    scores_ref[...] = s
    pred_ref[...] = pred


def _obj_call(obj_logits):
    blk = 2048
    grid = (N_OBJ + blk - 1) // blk
    return pl.pallas_call(
        _obj_body,
        grid=(grid,),
        in_specs=[pl.BlockSpec((blk, O_CLS), lambda i: (i, 0))],
        out_specs=[
            pl.BlockSpec((blk, O_CLS), lambda i: (i, 0)),
            pl.BlockSpec((blk,), lambda i: (i,)),
            pl.BlockSpec((blk,), lambda i: (i,)),
        ],
        out_shape=[
            jax.ShapeDtypeStruct((N_OBJ, O_CLS), jnp.float32),
            jax.ShapeDtypeStruct((N_OBJ,), jnp.float32),
            jax.ShapeDtypeStruct((N_OBJ,), jnp.int32),
        ],
    )(obj_logits)


def _rel_body(logits_ref, pairs_ref, packed_ref, scores_ref, cls_ref,
              i0_ref, i1_ref):
    x = logits_ref[...]
    rows = x.shape[0]
    m = jnp.max(x, axis=1, keepdims=True)
    e = jnp.exp(x - m)
    z = _xla_row_sum(e, 128)
    p = e / z
    i0 = pairs_ref[:, 0:1]
    i1 = pairs_ref[:, 1:2]
    packed_ref[:, :R_CLS] = p
    packed_ref[:, R_CLS : R_CLS + 1] = lax.bitcast_convert_type(i0, jnp.float32)
    packed_ref[:, R_CLS + 1 : R_CLS + 2] = lax.bitcast_convert_type(
        i1, jnp.float32
    )
    packed_ref[:, R_CLS + 2 :] = jnp.zeros(
        (rows, PACK_W - R_CLS - 2), jnp.float32
    )
    pfg = p[:, : R_CLS - 1]
    smax = jnp.max(pfg, axis=1, keepdims=True)
    scores_ref[...] = smax[:, 0]
    col = lax.broadcasted_iota(jnp.int32, pfg.shape, 1)
    cls_ref[...] = jnp.min(jnp.where(pfg >= smax, col, jnp.int32(2**30)), axis=1)
    i0_ref[...] = i0[:, 0]
    i1_ref[...] = i1[:, 0]


def _rel_call(rel_logits, rel_pair_idx):
    blk = 2048
    grid = (N_REL + blk - 1) // blk
    return pl.pallas_call(
        _rel_body,
        grid=(grid,),
        in_specs=[
            pl.BlockSpec((blk, R_CLS), lambda i: (i, 0)),
            pl.BlockSpec((blk, 2), lambda i: (i, 0)),
        ],
        out_specs=[
            pl.BlockSpec((blk, PACK_W), lambda i: (i, 0)),
            pl.BlockSpec((blk,), lambda i: (i,)),
            pl.BlockSpec((blk,), lambda i: (i,)),
            pl.BlockSpec((blk,), lambda i: (i,)),
            pl.BlockSpec((blk,), lambda i: (i,)),
        ],
        out_shape=[
            jax.ShapeDtypeStruct((N_REL, PACK_W), jnp.float32),
            jax.ShapeDtypeStruct((N_REL,), jnp.float32),
            jax.ShapeDtypeStruct((N_REL,), jnp.int32),
            jax.ShapeDtypeStruct((N_REL,), jnp.int32),
            jax.ShapeDtypeStruct((N_REL,), jnp.int32),
        ],
    )(rel_logits, rel_pair_idx)


def _triple_body(i0_hbm, i1_hbm, s_hbm, obj_hbm, out_hbm,
                 i0v, i1v, sbuf, s0b, s1b, tbuf, sem0, sem1):
    w = lax.axis_index("s") * SC_NC + lax.axis_index("c")
    base = w * PER_W
    pltpu.sync_copy(i0_hbm.at[pl.ds(w * NCH_P, NCH_P)], i0v)
    pltpu.sync_copy(i1_hbm.at[pl.ds(w * NCH_P, NCH_P)], i1v)
    pltpu.sync_copy(s_hbm.at[pl.ds(base, PER_W)], sbuf)

    def step(c, _):
        d0 = pltpu.async_copy(obj_hbm.at[i0v.at[c]], s0b, sem0)
        d1 = pltpu.async_copy(obj_hbm.at[i1v.at[c]], s1b, sem1)
        d0.wait()
        d1.wait()

        def vstep(v, _):
            o = c * CH + v * 16
            tbuf[pl.ds(o, 16)] = (
                sbuf[pl.ds(o, 16)]
                * s0b[pl.ds(v * 16, 16)]
                * s1b[pl.ds(v * 16, 16)]
            )
            return 0

        lax.fori_loop(0, CH // 16, vstep, 0)
        return 0

    lax.fori_loop(0, NCH, step, 0)
    pltpu.sync_copy(tbuf, out_hbm.at[pl.ds(base, PER_W)])


def _triple_call(obj_scores, rel_scores_p, i0r, i1r):
    mesh = plsc.VectorSubcoreMesh(
        core_axis_name="c", subcore_axis_name="s",
        num_cores=SC_NC, num_subcores=SC_NS,
    )
    fn = pl.kernel(
        _triple_body,
        out_type=jax.ShapeDtypeStruct((REL_PAD,), jnp.float32),
        mesh=mesh,
        scratch_types=[
            pltpu.VMEM((NCH_P, CH), jnp.int32),
            pltpu.VMEM((NCH_P, CH), jnp.int32),
            pltpu.VMEM((PER_W,), jnp.float32),
            pltpu.VMEM((CH,), jnp.float32),
            pltpu.VMEM((CH,), jnp.float32),
            pltpu.VMEM((PER_W,), jnp.float32),
            pltpu.SemaphoreType.DMA,
            pltpu.SemaphoreType.DMA,
        ],
    )
    return fn(i0r, i1r, rel_scores_p, obj_scores)


def _make_sort_tables():
    js, ks = [], []
    k = 2
    while k <= SORT_N:
        j = k // 2
        while j > 0:
            js.append(j)
            ks.append(k)
            j //= 2
        k *= 2
    return js, ks


def _sort_body(jt_ref, kt_ref, keys_any, out_any, ks, isc, ia, sem):
    p = pl.program_id(0)
    npass = pl.num_programs(0)

    @pl.when(p == 0)
    def _init():
        pltpu.make_async_copy(keys_any, ks, sem).start()
        ia[...] = (
            lax.broadcasted_iota(jnp.int32, (SORT_R, SORT_C), 0) * SORT_C
            + lax.broadcasted_iota(jnp.int32, (SORT_R, SORT_C), 1)
        )
        pltpu.make_async_copy(keys_any, ks, sem).wait()
        isc[...] = ia[...]

    j = jt_ref[p]
    k = kt_ref[p]
    kv = ks[...]
    iv = isc[...]
    iav = ia[...]
    low = (iav & j) == 0
    asc = (iav & k) == 0

    def lane_case(_):
        kp = jnp.where(
            low,
            pltpu.roll(kv, SORT_C - j, axis=1),
            pltpu.roll(kv, j, axis=1),
        )
        ip = jnp.where(
            low,
            pltpu.roll(iv, SORT_C - j, axis=1),
            pltpu.roll(iv, j, axis=1),
        )
        return kp, ip

    def row_case(_):
        jr = j // SORT_C
        kp = jnp.where(
            low,
            pltpu.roll(kv, SORT_R - jr, axis=0),
            pltpu.roll(kv, jr, axis=0),
        )
        ip = jnp.where(
            low,
            pltpu.roll(iv, SORT_R - jr, axis=0),
            pltpu.roll(iv, jr, axis=0),
        )
        return kp, ip

    kp, ip = lax.cond(j < SORT_C, lane_case, row_case, operand=None)

    prec = (kv > kp) | ((kv == kp) & (iv < ip))
    wantmin = low == asc
    take = prec == wantmin
    ks[...] = jnp.where(take, kv, kp)
    isc[...] = jnp.where(take, iv, ip)

    @pl.when(p == npass - 1)
    def _fin():
        pltpu.make_async_copy(isc, out_any, sem).start()
        pltpu.make_async_copy(isc, out_any, sem).wait()


def _sort_call(keys_i2d):
    js, ks = _make_sort_tables()
    npass = len(js)
    jt = jnp.asarray(js, dtype=jnp.int32)
    kt = jnp.asarray(ks, dtype=jnp.int32)
    return pl.pallas_call(
        _sort_body,
        grid=(npass,),
        in_specs=[
            pl.BlockSpec(memory_space=pltpu.SMEM),
            pl.BlockSpec(memory_space=pltpu.SMEM),
            pl.BlockSpec(memory_space=pl.ANY),
        ],
        out_specs=pl.BlockSpec(memory_space=pl.ANY),
        out_shape=jax.ShapeDtypeStruct((SORT_R, SORT_C), jnp.int32),
        scratch_shapes=[
            pltpu.VMEM((SORT_R, SORT_C), jnp.int32),
            pltpu.VMEM((SORT_R, SORT_C), jnp.int32),
            pltpu.VMEM((SORT_R, SORT_C), jnp.int32),
            pltpu.SemaphoreType.DMA,
        ],
        compiler_params=pltpu.CompilerParams(
            dimension_semantics=("arbitrary",),
        ),
    )(jt, kt, keys_i2d)


def _gather_body(sidx_hbm, packed_hbm, cls_hbm, packed_out, cls_out,
                 idxv, rowb, labb, sem0, sem1):
    w = lax.axis_index("s") * SC_NC + lax.axis_index("c")

    pltpu.sync_copy(sidx_hbm.at[pl.ds(w * G_NCH_P, G_NCH_P)], idxv)

    def step(c, _):
        ix = idxv.at[c]
        d0 = pltpu.async_copy(packed_hbm.at[ix], rowb, sem0)
        d1 = pltpu.async_copy(cls_hbm.at[ix], labb, sem1)
        d0.wait()
        d1.wait()
        o = w * PER_W + c * G_CH
        pltpu.sync_copy(rowb, packed_out.at[pl.ds(o, G_CH)])
        pltpu.sync_copy(labb, cls_out.at[pl.ds(o, G_CH)])
        return 0

    lax.fori_loop(0, G_NCH, step, 0)


def _gather_call(sidx2d, packed, rel_class):
    mesh = plsc.VectorSubcoreMesh(
        core_axis_name="c", subcore_axis_name="s",
        num_cores=SC_NC, num_subcores=SC_NS,
    )
    fn = pl.kernel(
        _gather_body,
        out_type=[
            jax.ShapeDtypeStruct((REL_PAD, PACK_W), jnp.float32),
            jax.ShapeDtypeStruct((REL_PAD,), jnp.int32),
        ],
        mesh=mesh,
        scratch_types=[
            pltpu.VMEM((G_NCH_P, G_CH), jnp.int32),
            pltpu.VMEM((G_CH, PACK_W), jnp.float32),
            pltpu.VMEM((G_CH,), jnp.int32),
            pltpu.SemaphoreType.DMA,
            pltpu.SemaphoreType.DMA,
        ],
    )
    return fn(sidx2d, packed, rel_class)


def _unpack_body(packed_ref, prob_ref, pairs_ref):
    blkrow = packed_ref[...]
    prob_ref[...] = blkrow[:, :R_CLS]
    pairs_ref[...] = lax.bitcast_convert_type(
        blkrow[:, R_CLS : R_CLS + 2], jnp.int32
    )


def _unpack_call(packed_s):
    blk = 4096
    grid = REL_PAD // blk
    return pl.pallas_call(
        _unpack_body,
        grid=(grid,),
        in_specs=[pl.BlockSpec((blk, PACK_W), lambda i: (i, 0))],
        out_specs=[
            pl.BlockSpec((blk, R_CLS), lambda i: (i, 0)),
            pl.BlockSpec((blk, 2), lambda i: (i, 0)),
        ],
        out_shape=[
            jax.ShapeDtypeStruct((N_REL, R_CLS), jnp.float32),
            jax.ShapeDtypeStruct((N_REL, 2), jnp.int32),
        ],
    )(packed_s)


def kernel(rel_logits, obj_logits, rel_pair_idx, box, img_size):
    obj_class_prob, obj_scores, obj_pred = _obj_call(obj_logits)
    packed, rel_scores, rel_class, i0, i1 = _rel_call(rel_logits, rel_pair_idx)

    pad = REL_PAD - N_REL
    rel_scores_p = jnp.pad(rel_scores, (0, pad))
    def _chunk_rows(x, nch, nch_p, ch):
        x = x.reshape(SC_NW, nch * ch)
        x = jnp.pad(x, ((0, 0), (0, (nch_p - nch) * ch)))
        return x.reshape(SC_NW * nch_p, ch)

    i0r = _chunk_rows(jnp.pad(i0, (0, pad)), NCH, NCH_P, CH)
    i1r = _chunk_rows(jnp.pad(i1, (0, pad)), NCH, NCH_P, CH)
    triple_p = _triple_call(obj_scores, rel_scores_p, i0r, i1r)

    keys = jnp.pad(triple_p, (0, SORT_N - REL_PAD))
    keys_i2d = lax.bitcast_convert_type(keys, jnp.int32).reshape(
        SORT_R, SORT_C
    )
    sidx = _sort_call(keys_i2d).reshape(-1)[:N_REL]
    sidx2d = _chunk_rows(jnp.pad(sidx, (0, pad)), G_NCH, G_NCH_P, G_CH)

    packed_s, labels_p = _gather_call(sidx2d, packed, rel_class)
    rel_prob_s, pairs_s = _unpack_call(packed_s)
    labels_s = labels_p[:N_REL]

    return (box, obj_pred, obj_scores, obj_class_prob,
            pairs_s, rel_prob_s, labels_s)
